# Initial kernel scaffold; baseline (speedup 1.0000x reference)
import jax
import jax.numpy as jnp
from jax.experimental import pallas as pl


def kernel(x_vecs, device_weights, device_bias, category_weights, category_bias):
    raise NotImplementedError("write your pallas kernel here")


# speedup vs baseline: 2.4437x; 2.4437x over previous
import functools

import jax
import jax.numpy as jnp
from jax import lax
from jax.experimental import pallas as pl
from jax.experimental.pallas import tpu as pltpu
from jax.experimental.pallas import tpu_sc as plsc

B, R, H, W = 128, 34, 512, 512
NUM_CATS = 5
CHW = 2 * H * W
TOTAL = B * CHW

NROWS = 2 * R
IMG_ROWS = 32768
IMG_COLS = 2048
BLK_ROWS = 512
GRID = IMG_ROWS // BLK_ROWS


def _prep_memset_kernel(xt_ref, dw_ref, db_ref, cw_ref, cb_ref,
                        img_ref, vals_ref, idx_ref):
  img_ref[...] = jnp.zeros((BLK_ROWS, IMG_COLS), jnp.float32)

  @pl.when(pl.program_id(0) == 0)
  def _prep():
    pw = xt_ref[0]
    cx = xt_ref[1]
    cy = xt_ref[2]
    cat = xt_ref[3].astype(jnp.int32)
    pind = (pw != 0.0).astype(jnp.float32)
    proc = pw * dw_ref[...] + pind * db_ref[...]
    cwv = jnp.zeros((R, B), jnp.float32)
    cbv = jnp.zeros((R, B), jnp.float32)
    for c in range(NUM_CATS):
      m = cat == c
      cwv = jnp.where(m, cw_ref[0, c], cwv)
      cbv = jnp.where(m, cb_ref[0, c], cbv)
    proc = proc * cwv + pind * cbv

    xi = jnp.round(cx).astype(jnp.int32)
    yi = jnp.round(cy).astype(jnp.int32)
    key = yi * W + xi
    bcol = lax.broadcasted_iota(jnp.int32, (R, B), 1)

    vproc, vraw = proc, pw
    for rp in range(R):
      m = key == key[rp:rp + 1]
      vproc = jnp.where(m, proc[rp:rp + 1], vproc)
      vraw = jnp.where(m, pw[rp:rp + 1], vraw)
    vals_ref[0] = vproc
    vals_ref[1] = vraw
    idx_ref[0] = bcol * CHW + key
    idx_ref[1] = bcol * CHW + H * W + key


def _sc_scatter(img_ref, vals_hbm, idx_hbm, vals_v, idx_v, sem):
  info = plsc.get_sparse_core_info()
  nc = info.num_cores
  wid = lax.axis_index("s") * nc + lax.axis_index("c")
  for j in range(3):
    row = wid + 32 * j

    @pl.when(row < NROWS)
    def _do():
      pltpu.sync_copy(idx_hbm.at[row], idx_v)
      pltpu.sync_copy(vals_hbm.at[row], vals_v)
      pltpu.async_copy(vals_v, img_ref.at[idx_v], sem).wait()


@functools.cache
def _sc_scatter_call():
  return pl.kernel(
      _sc_scatter,
      out_type=(),
      mesh=plsc.VectorSubcoreMesh(core_axis_name="c", subcore_axis_name="s"),
      scratch_types=[
          pltpu.VMEM((B,), jnp.float32),
          pltpu.VMEM((B,), jnp.int32),
          pltpu.SemaphoreType.DMA,
      ],
  )


@jax.jit
def kernel(x_vecs, device_weights, device_bias, category_weights,
           category_bias):
  xt = x_vecs.transpose(2, 1, 0)
  dwb = jnp.broadcast_to(device_weights[:, None], (R, B))
  dbb = jnp.broadcast_to(device_bias[:, None], (R, B))
  cw2 = category_weights.reshape(1, NUM_CATS)
  cb2 = category_bias.reshape(1, NUM_CATS)

  img2d, vals, idx = pl.pallas_call(
      _prep_memset_kernel,
      grid=(GRID,),
      in_specs=[
          pl.BlockSpec((4, R, B), lambda i: (0, 0, 0)),
          pl.BlockSpec((R, B), lambda i: (0, 0)),
          pl.BlockSpec((R, B), lambda i: (0, 0)),
          pl.BlockSpec((1, NUM_CATS), lambda i: (0, 0),
                       memory_space=pltpu.SMEM),
          pl.BlockSpec((1, NUM_CATS), lambda i: (0, 0),
                       memory_space=pltpu.SMEM),
      ],
      out_specs=[
          pl.BlockSpec((BLK_ROWS, IMG_COLS), lambda i: (i, 0)),
          pl.BlockSpec((2, R, B), lambda i: (0, 0, 0)),
          pl.BlockSpec((2, R, B), lambda i: (0, 0, 0)),
      ],
      out_shape=[
          jax.ShapeDtypeStruct((IMG_ROWS, IMG_COLS), jnp.float32),
          jax.ShapeDtypeStruct((2, R, B), jnp.float32),
          jax.ShapeDtypeStruct((2, R, B), jnp.int32),
      ],
  )(xt, dwb, dbb, cw2, cb2)

  img_ref = jax.new_ref(img2d.reshape(TOTAL))
  _sc_scatter_call()(img_ref, vals.reshape(NROWS, B), idx.reshape(NROWS, B))
  return img_ref[...].reshape(B, 2, H, W)
